# baseline (speedup 1.0000x reference)
import functools

import jax
import jax.numpy as jnp
import numpy as np
from jax import lax
from jax.experimental import pallas as pl
from jax.experimental.pallas import tpu as pltpu
from jax.experimental.pallas import tpu_sc as plsc

B = 16384
V = 100000
F = 26
D = 32
P = 64
NNUM = 13
NF = F + 1
NPAIR = NF * (NF - 1) // 2
NROLL = NF // 2
IC = NF * P

_GW = 128
_BT = 512
FP = 28


def _leaky(x):
    return jnp.where(x >= 0, x, 0.01 * x)


@jax.jit
def _sc_gather(tables_flat, flat_idx):
    mesh = plsc.VectorSubcoreMesh(core_axis_name="core", subcore_axis_name="subcore")

    @functools.partial(
        pl.kernel,
        out_type=jax.ShapeDtypeStruct((B * F, D * 4), jnp.uint8),
        mesh=mesh,
        compiler_params=pltpu.CompilerParams(use_tc_tiling_on_sc=False),
    )
    def gk(x_hbm, i_hbm, o_hbm):
        def body(i_vmem, o_vmem):
            pltpu.sync_copy(x_hbm.at[i_vmem.at[0]], o_vmem)

        pltpu.emit_pipeline(
            body,
            grid=(B * F // _GW,),
            in_specs=[pl.BlockSpec((1, _GW), index_map=lambda i: (0, i))],
            out_specs=[pl.BlockSpec((_GW, D * 4), index_map=lambda i: (i, 0))],
            core_axis_name=("core", "subcore"),
            dimension_semantics=(pltpu.PARALLEL,),
        )(i_hbm, o_hbm)

    return gk(tables_flat, flat_idx)


def _dense_body(num_ref, e_ref, bW0_ref, bb0_ref, bW1_ref, bb1_ref, bW2_ref,
                bb2_ref, wpk_ref, pb_ref, sseg_ref, w0c_ref, w0n_ref, tb0_ref,
                tW1_ref, tb1_ref, tW2_ref, tb2_ref, tW3_ref, tb3_ref, o_ref):
    f32 = jnp.float32
    mm = lambda a, b: jnp.dot(a, b, preferred_element_type=f32)

    x = num_ref[...]
    x = _leaky(mm(x.astype(jnp.bfloat16), bW0_ref[...]) + bb0_ref[...])
    x = _leaky(mm(x.astype(jnp.bfloat16), bW1_ref[...]) + bb1_ref[...])
    x = _leaky(mm(x.astype(jnp.bfloat16), bW2_ref[...]) + bb2_ref[...])

    e = e_ref[...].astype(jnp.bfloat16)
    ys = []
    for g in range(6):
        ys.append(mm(e[:, g * 128:(g + 1) * 128], wpk_ref[g]))
    ys.append(mm(e[:, 768:832], wpk_ref[6][:64, :128]))
    y = jnp.concatenate(ys, axis=1) + pb_ref[...]

    ic = jnp.concatenate([x, y], axis=1).astype(jnp.bfloat16)

    dots = []
    for o in range(1, NROLL + 1):
        rolled = jnp.concatenate([ic[:, o * P:], ic[:, :o * P]], axis=1)
        prod = ic * rolled
        dots.append(mm(prod, sseg_ref[...]))
    dotcat = jnp.concatenate(dots, axis=1)

    h = mm(dotcat.astype(jnp.bfloat16), w0c_ref[...])
    h = h + mm(num_ref[...].astype(jnp.bfloat16), w0n_ref[...]) + tb0_ref[...]
    h = _leaky(h)
    h = _leaky(mm(h.astype(jnp.bfloat16), tW1_ref[...]) + tb1_ref[...])
    h = _leaky(mm(h.astype(jnp.bfloat16), tW2_ref[...]) + tb2_ref[...])
    h = mm(h.astype(jnp.bfloat16), tW3_ref[...]) + tb3_ref[...]
    o_ref[...] = 1.0 / (1.0 + jnp.exp(-h))


@jax.jit
def _tc_dense(num, e2d, bW0, bb0, bW1, bb1, bW2, bb2, wpk, pb, sseg, w0c, w0n,
              tb0, tW1, tb1, tW2, tb2, tW3, tb3):
    rep = lambda s: pl.BlockSpec(s, lambda i: tuple(0 for _ in s))
    grid = (B // _BT,)
    return pl.pallas_call(
        _dense_body,
        grid=grid,
        in_specs=[
            pl.BlockSpec((_BT, NNUM), lambda i: (i, 0)),
            pl.BlockSpec((_BT, F * D), lambda i: (i, 0)),
            rep((NNUM, 512)), rep((1, 512)),
            rep((512, 256)), rep((1, 256)),
            rep((256, 64)), rep((1, 64)),
            rep((7, 128, 256)), rep((1, F * P)),
            rep((IC, NF)),
            rep((NPAIR, 1024)), rep((NNUM, 1024)), rep((1, 1024)),
            rep((1024, 512)), rep((1, 512)),
            rep((512, 256)), rep((1, 256)),
            rep((256, 1)), rep((1, 1)),
        ],
        out_specs=pl.BlockSpec((_BT, 1), lambda i: (i, 0)),
        out_shape=jax.ShapeDtypeStruct((B, 1), jnp.float32),
    )(num, e2d, bW0, bb0, bW1, bb1, bW2, bb2, wpk, pb, sseg, w0c, w0n,
      tb0, tW1, tb1, tW2, tb2, tW3, tb3)


def _pair_index(a, b):
    return a * (2 * NF - a - 1) // 2 + (b - a - 1)


def kernel(num, cat, tables, proj_W, proj_b, bW0, bb0, bW1, bb1, bW2, bb2,
           tW0, tb0, tW1, tb1, tW2, tb2, tW3, tb3):
    bf = jnp.bfloat16

    tables_u8 = jax.lax.bitcast_convert_type(tables, jnp.uint8).reshape(
        F * V, D * 4)
    flat_idx = (cat.astype(jnp.int32)
                + (jnp.arange(F, dtype=jnp.int32) * V)[None, :]).reshape(1, B * F)
    e_u8 = _sc_gather(tables_u8, flat_idx)
    e2d = jax.lax.bitcast_convert_type(
        e_u8.reshape(B * F, D, 4), jnp.float32).reshape(B, F * D)

    pw4 = jnp.concatenate(
        [proj_W, jnp.zeros((28 - F, D, P), proj_W.dtype)], axis=0
    ).reshape(7, 4, D, P)
    eye4 = jnp.asarray(np.eye(4, dtype=np.float32))
    wpk = jnp.einsum('gadp,ab->gadbp', pw4, eye4).reshape(7, 128, 256).astype(bf)
    pb = proj_b.reshape(1, F * P).astype(jnp.float32)

    sseg = jnp.asarray(np.repeat(np.eye(NF, dtype=np.float32), P, axis=0),
                       dtype=bf)

    rows = []
    for o in range(1, NROLL + 1):
        for n in range(NF):
            m = n + o
            a, b = (n, m) if m < NF else (m - NF, n)
            rows.append(_pair_index(a, b))
    w0c = tW0[jnp.array(rows, dtype=jnp.int32)].astype(bf)
    w0n = tW0[NPAIR:].astype(bf)

    out = _tc_dense(
        num, e2d,
        bW0.astype(bf), bb0.reshape(1, -1), bW1.astype(bf), bb1.reshape(1, -1),
        bW2.astype(bf), bb2.reshape(1, -1), wpk, pb, sseg, w0c, w0n,
        tb0.reshape(1, -1), tW1.astype(bf), tb1.reshape(1, -1),
        tW2.astype(bf), tb2.reshape(1, -1), tW3.astype(bf), tb3.reshape(1, -1))
    return jnp.squeeze(out, axis=1)


# speedup vs baseline: 29.6782x; 29.6782x over previous
import functools

import jax
import jax.numpy as jnp
import numpy as np
from jax import lax
from jax.experimental import pallas as pl
from jax.experimental.pallas import tpu as pltpu
from jax.experimental.pallas import tpu_sc as plsc

B = 16384
V = 100000
F = 26
D = 32
P = 64
NNUM = 13
NF = F + 1
NPAIR = NF * (NF - 1) // 2
NROLL = NF // 2
IC = NF * P

_GW = 128
_BT = 512
FP = 28


def _leaky(x):
    return jnp.where(x >= 0, x, 0.01 * x)


@jax.jit
def _sc_gather(tables_flat, flat_idx):
    mesh = plsc.VectorSubcoreMesh(core_axis_name="core", subcore_axis_name="subcore")

    @functools.partial(
        pl.kernel,
        out_type=jax.ShapeDtypeStruct((B * F, D), jnp.float32),
        mesh=mesh,
        compiler_params=pltpu.CompilerParams(use_tc_tiling_on_sc=False),
    )
    def gk(x_hbm, i_hbm, o_hbm):
        def body(i_vmem, o_vmem):
            pltpu.sync_copy(x_hbm.at[i_vmem.at[0]], o_vmem)

        pltpu.emit_pipeline(
            body,
            grid=(B * F // _GW,),
            in_specs=[pl.BlockSpec((1, _GW), index_map=lambda i: (0, i))],
            out_specs=[pl.BlockSpec((_GW, D), index_map=lambda i: (i, 0))],
            core_axis_name=("core", "subcore"),
            dimension_semantics=(pltpu.PARALLEL,),
        )(i_hbm, o_hbm)

    return gk(tables_flat, flat_idx)


def _dense_body(num_ref, e_ref, bW0_ref, bb0_ref, bW1_ref, bb1_ref, bW2_ref,
                bb2_ref, wpk_ref, pb_ref, sseg_ref, w0c_ref, w0n_ref, tb0_ref,
                tW1_ref, tb1_ref, tW2_ref, tb2_ref, tW3_ref, tb3_ref, o_ref):
    f32 = jnp.float32
    mm = lambda a, b: jnp.dot(a, b, preferred_element_type=f32)

    x = num_ref[...]
    x = _leaky(mm(x.astype(jnp.bfloat16), bW0_ref[...]) + bb0_ref[...])
    x = _leaky(mm(x.astype(jnp.bfloat16), bW1_ref[...]) + bb1_ref[...])
    x = _leaky(mm(x.astype(jnp.bfloat16), bW2_ref[...]) + bb2_ref[...])

    e = e_ref[...].astype(jnp.bfloat16)
    ys = []
    for g in range(6):
        ys.append(mm(e[:, g * 128:(g + 1) * 128], wpk_ref[g]))
    ys.append(mm(e[:, 768:832], wpk_ref[6][:64, :128]))
    y = jnp.concatenate(ys, axis=1) + pb_ref[...]

    ic = jnp.concatenate([x, y], axis=1).astype(jnp.bfloat16)

    dots = []
    for o in range(1, NROLL + 1):
        rolled = jnp.concatenate([ic[:, o * P:], ic[:, :o * P]], axis=1)
        prod = ic * rolled
        dots.append(mm(prod, sseg_ref[...]))
    dotcat = jnp.concatenate(dots, axis=1)

    h = mm(dotcat.astype(jnp.bfloat16), w0c_ref[...])
    h = h + mm(num_ref[...].astype(jnp.bfloat16), w0n_ref[...]) + tb0_ref[...]
    h = _leaky(h)
    h = _leaky(mm(h.astype(jnp.bfloat16), tW1_ref[...]) + tb1_ref[...])
    h = _leaky(mm(h.astype(jnp.bfloat16), tW2_ref[...]) + tb2_ref[...])
    h = mm(h.astype(jnp.bfloat16), tW3_ref[...]) + tb3_ref[...]
    o_ref[...] = 1.0 / (1.0 + jnp.exp(-h))


@jax.jit
def _tc_dense(num, e2d, bW0, bb0, bW1, bb1, bW2, bb2, wpk, pb, sseg, w0c, w0n,
              tb0, tW1, tb1, tW2, tb2, tW3, tb3):
    rep = lambda s: pl.BlockSpec(s, lambda i: tuple(0 for _ in s))
    grid = (B // _BT,)
    return pl.pallas_call(
        _dense_body,
        grid=grid,
        in_specs=[
            pl.BlockSpec((_BT, NNUM), lambda i: (i, 0)),
            pl.BlockSpec((_BT, F * D), lambda i: (i, 0)),
            rep((NNUM, 512)), rep((1, 512)),
            rep((512, 256)), rep((1, 256)),
            rep((256, 64)), rep((1, 64)),
            rep((7, 128, 256)), rep((1, F * P)),
            rep((IC, NF)),
            rep((NPAIR, 1024)), rep((NNUM, 1024)), rep((1, 1024)),
            rep((1024, 512)), rep((1, 512)),
            rep((512, 256)), rep((1, 256)),
            rep((256, 1)), rep((1, 1)),
        ],
        out_specs=pl.BlockSpec((_BT, 1), lambda i: (i, 0)),
        out_shape=jax.ShapeDtypeStruct((B, 1), jnp.float32),
    )(num, e2d, bW0, bb0, bW1, bb1, bW2, bb2, wpk, pb, sseg, w0c, w0n,
      tb0, tW1, tb1, tW2, tb2, tW3, tb3)


def _pair_index(a, b):
    return a * (2 * NF - a - 1) // 2 + (b - a - 1)


def kernel(num, cat, tables, proj_W, proj_b, bW0, bb0, bW1, bb1, bW2, bb2,
           tW0, tb0, tW1, tb1, tW2, tb2, tW3, tb3):
    bf = jnp.bfloat16

    tables_flat = tables.reshape(F * V, D)
    flat_idx = (cat.astype(jnp.int32)
                + (jnp.arange(F, dtype=jnp.int32) * V)[None, :]).reshape(1, B * F)
    e2d = _sc_gather(tables_flat, flat_idx).reshape(B, F * D)

    pw4 = jnp.concatenate(
        [proj_W, jnp.zeros((28 - F, D, P), proj_W.dtype)], axis=0
    ).reshape(7, 4, D, P)
    eye4 = jnp.asarray(np.eye(4, dtype=np.float32))
    wpk = jnp.einsum('gadp,ab->gadbp', pw4, eye4).reshape(7, 128, 256).astype(bf)
    pb = proj_b.reshape(1, F * P).astype(jnp.float32)

    sseg = jnp.asarray(np.repeat(np.eye(NF, dtype=np.float32), P, axis=0),
                       dtype=bf)

    rows = []
    for o in range(1, NROLL + 1):
        for n in range(NF):
            m = n + o
            a, b = (n, m) if m < NF else (m - NF, n)
            rows.append(_pair_index(a, b))
    w0c = tW0[jnp.array(rows, dtype=jnp.int32)].astype(bf)
    w0n = tW0[NPAIR:].astype(bf)

    out = _tc_dense(
        num, e2d,
        bW0.astype(bf), bb0.reshape(1, -1), bW1.astype(bf), bb1.reshape(1, -1),
        bW2.astype(bf), bb2.reshape(1, -1), wpk, pb, sseg, w0c, w0n,
        tb0.reshape(1, -1), tW1.astype(bf), tb1.reshape(1, -1),
        tW2.astype(bf), tb2.reshape(1, -1), tW3.astype(bf), tb3.reshape(1, -1))
    return jnp.squeeze(out, axis=1)


# speedup vs baseline: 30.0258x; 1.0117x over previous
import functools

import jax
import jax.numpy as jnp
import numpy as np
from jax import lax
from jax.experimental import pallas as pl
from jax.experimental.pallas import tpu as pltpu
from jax.experimental.pallas import tpu_sc as plsc

B = 16384
V = 100000
F = 26
D = 32
P = 64
NNUM = 13
NF = F + 1
NPAIR = NF * (NF - 1) // 2
NROLL = NF // 2
IC = NF * P

_GW = 128
_BT = 1024
FP = 28


def _leaky(x):
    return jnp.where(x >= 0, x, 0.01 * x)


@jax.jit
def _sc_gather(tables_flat, flat_idx):
    mesh = plsc.VectorSubcoreMesh(core_axis_name="core", subcore_axis_name="subcore")

    @functools.partial(
        pl.kernel,
        out_type=jax.ShapeDtypeStruct((B * F, D), jnp.float32),
        mesh=mesh,
        compiler_params=pltpu.CompilerParams(use_tc_tiling_on_sc=False),
    )
    def gk(x_hbm, i_hbm, o_hbm):
        def body(i_vmem, o_vmem):
            pltpu.sync_copy(x_hbm.at[i_vmem.at[0]], o_vmem)

        pltpu.emit_pipeline(
            body,
            grid=(B * F // _GW,),
            in_specs=[pl.BlockSpec((1, _GW), index_map=lambda i: (0, i))],
            out_specs=[pl.BlockSpec((_GW, D), index_map=lambda i: (i, 0))],
            core_axis_name=("core", "subcore"),
            dimension_semantics=(pltpu.PARALLEL,),
        )(i_hbm, o_hbm)

    return gk(tables_flat, flat_idx)


def _dense_body(num_ref, e_ref, bW0_ref, bb0_ref, bW1_ref, bb1_ref, bW2_ref,
                bb2_ref, wpk_ref, pb_ref, sseg_ref, w0c_ref, w0n_ref, tb0_ref,
                tW1_ref, tb1_ref, tW2_ref, tb2_ref, tW3_ref, tb3_ref, o_ref):
    f32 = jnp.float32
    mm = lambda a, b: jnp.dot(a, b, preferred_element_type=f32)

    x = num_ref[...]
    x = _leaky(mm(x.astype(jnp.bfloat16), bW0_ref[...]) + bb0_ref[...])
    x = _leaky(mm(x.astype(jnp.bfloat16), bW1_ref[...]) + bb1_ref[...])
    x = _leaky(mm(x.astype(jnp.bfloat16), bW2_ref[...]) + bb2_ref[...])

    e = e_ref[...].astype(jnp.bfloat16)
    ys = []
    for g in range(6):
        ys.append(mm(e[:, g * 128:(g + 1) * 128], wpk_ref[g]))
    ys.append(mm(e[:, 768:832], wpk_ref[6][:64, :128]))
    y = jnp.concatenate(ys, axis=1) + pb_ref[...]

    ic = jnp.concatenate([x, y], axis=1).astype(jnp.bfloat16)

    dots = []
    for o in range(1, NROLL + 1):
        rolled = jnp.concatenate([ic[:, o * P:], ic[:, :o * P]], axis=1)
        prod = ic * rolled
        dots.append(mm(prod, sseg_ref[...]))
    dotcat = jnp.concatenate(dots, axis=1)

    h = mm(dotcat.astype(jnp.bfloat16), w0c_ref[...])
    h = h + mm(num_ref[...].astype(jnp.bfloat16), w0n_ref[...]) + tb0_ref[...]
    h = _leaky(h)
    h = _leaky(mm(h.astype(jnp.bfloat16), tW1_ref[...]) + tb1_ref[...])
    h = _leaky(mm(h.astype(jnp.bfloat16), tW2_ref[...]) + tb2_ref[...])
    h = mm(h.astype(jnp.bfloat16), tW3_ref[...]) + tb3_ref[...]
    o_ref[...] = 1.0 / (1.0 + jnp.exp(-h))


@jax.jit
def _tc_dense(num, e2d, bW0, bb0, bW1, bb1, bW2, bb2, wpk, pb, sseg, w0c, w0n,
              tb0, tW1, tb1, tW2, tb2, tW3, tb3):
    rep = lambda s: pl.BlockSpec(s, lambda i: tuple(0 for _ in s))
    grid = (B // _BT,)
    return pl.pallas_call(
        _dense_body,
        grid=grid,
        in_specs=[
            pl.BlockSpec((_BT, NNUM), lambda i: (i, 0)),
            pl.BlockSpec((_BT, F * D), lambda i: (i, 0)),
            rep((NNUM, 512)), rep((1, 512)),
            rep((512, 256)), rep((1, 256)),
            rep((256, 64)), rep((1, 64)),
            rep((7, 128, 256)), rep((1, F * P)),
            rep((IC, NF)),
            rep((NPAIR, 1024)), rep((NNUM, 1024)), rep((1, 1024)),
            rep((1024, 512)), rep((1, 512)),
            rep((512, 256)), rep((1, 256)),
            rep((256, 1)), rep((1, 1)),
        ],
        out_specs=pl.BlockSpec((_BT, 1), lambda i: (i, 0)),
        out_shape=jax.ShapeDtypeStruct((B, 1), jnp.float32),
    )(num, e2d, bW0, bb0, bW1, bb1, bW2, bb2, wpk, pb, sseg, w0c, w0n,
      tb0, tW1, tb1, tW2, tb2, tW3, tb3)


def _pair_index(a, b):
    return a * (2 * NF - a - 1) // 2 + (b - a - 1)


def kernel(num, cat, tables, proj_W, proj_b, bW0, bb0, bW1, bb1, bW2, bb2,
           tW0, tb0, tW1, tb1, tW2, tb2, tW3, tb3):
    bf = jnp.bfloat16

    tables_flat = tables.reshape(F * V, D)
    flat_idx = (cat.astype(jnp.int32)
                + (jnp.arange(F, dtype=jnp.int32) * V)[None, :]).reshape(1, B * F)
    e2d = _sc_gather(tables_flat, flat_idx).reshape(B, F * D)

    pw4 = jnp.concatenate(
        [proj_W, jnp.zeros((28 - F, D, P), proj_W.dtype)], axis=0
    ).reshape(7, 4, D, P)
    eye4 = jnp.asarray(np.eye(4, dtype=np.float32))
    wpk = jnp.einsum('gadp,ab->gadbp', pw4, eye4).reshape(7, 128, 256).astype(bf)
    pb = proj_b.reshape(1, F * P).astype(jnp.float32)

    sseg = jnp.asarray(np.repeat(np.eye(NF, dtype=np.float32), P, axis=0),
                       dtype=bf)

    rows = []
    for o in range(1, NROLL + 1):
        for n in range(NF):
            m = n + o
            a, b = (n, m) if m < NF else (m - NF, n)
            rows.append(_pair_index(a, b))
    w0c = tW0[jnp.array(rows, dtype=jnp.int32)].astype(bf)
    w0n = tW0[NPAIR:].astype(bf)

    out = _tc_dense(
        num, e2d,
        bW0.astype(bf), bb0.reshape(1, -1), bW1.astype(bf), bb1.reshape(1, -1),
        bW2.astype(bf), bb2.reshape(1, -1), wpk, pb, sseg, w0c, w0n,
        tb0.reshape(1, -1), tW1.astype(bf), tb1.reshape(1, -1),
        tW2.astype(bf), tb2.reshape(1, -1), tW3.astype(bf), tb3.reshape(1, -1))
    return jnp.squeeze(out, axis=1)
